# Initial kernel scaffold; baseline (speedup 1.0000x reference)
import jax
import jax.numpy as jnp
from jax.experimental import pallas as pl


def kernel(x, encodings):
    raise NotImplementedError("write your pallas kernel here")


# speedup vs baseline: 2.6216x; 2.6216x over previous
import jax
import jax.numpy as jnp
from jax.experimental import pallas as pl


def _copy_body(enc_ref, out_ref):
    out_ref[...] = enc_ref[...]


def kernel(x, encodings):
    seq, d = encodings.shape
    rows = 1024
    grid = (seq // rows,)
    out = pl.pallas_call(
        _copy_body,
        grid=grid,
        in_specs=[pl.BlockSpec((rows, d), lambda i: (i, 0))],
        out_specs=pl.BlockSpec((rows, d), lambda i: (i, 0)),
        out_shape=jax.ShapeDtypeStruct((seq, d), jnp.float32),
    )(encodings)
    return out[None, :, :]
